# baseline (speedup 1.0000x reference)
import functools

import jax
import jax.numpy as jnp
from jax import lax
from jax.experimental import pallas as pl
from jax.experimental.pallas import tpu as pltpu
from jax.experimental.pallas import tpu_sc as plsc

NC = 2
NS = 16
NW = NC * NS
LANES = 16


@functools.partial(jax.jit, static_argnums=(3, 4, 5))
def _sc_embed(ids, token_table, position_table, B, L, E):
    steps = B // NW
    n_ids = steps * L
    scale = 8.0

    mesh = plsc.VectorSubcoreMesh(core_axis_name="c", subcore_axis_name="s")

    @functools.partial(
        pl.kernel,
        out_type=jax.ShapeDtypeStruct((B * L // 2, 2 * E), jnp.float32),
        mesh=mesh,
        scratch_types=[
            pltpu.VMEM((n_ids,), jnp.int32),
            pltpu.VMEM((L, E), jnp.float32),
            pltpu.VMEM((L, E), jnp.float32),
            pltpu.VMEM((L, E), jnp.float32),
            pltpu.VMEM((L, E), jnp.float32),
            pltpu.VMEM((L, E), jnp.float32),
            pltpu.SemaphoreType.DMA,
            pltpu.SemaphoreType.DMA,
            pltpu.SemaphoreType.DMA,
            pltpu.SemaphoreType.DMA,
        ],
        compiler_params=pltpu.CompilerParams(use_tc_tiling_on_sc=False),
    )
    def k(table_hbm, ids_hbm, pos_hbm, out_hbm, ids_all, pos_v,
          rows0, rows1, outb0, outb1, sg0, sg1, so0, so1):
        rows_v = (rows0, rows1)
        out_v = (outb0, outb1)
        sg = (sg0, sg1)
        so = (so0, so1)

        wid = lax.axis_index("s") * NC + lax.axis_index("c")
        wbase = wid * n_ids

        pltpu.sync_copy(pos_hbm, pos_v)
        pltpu.sync_copy(ids_hbm.at[pl.ds(wbase, n_ids)], ids_all)

        g_chunks = [(o, min(128, L - o)) for o in range(0, L, 128)]

        def gather_fire(b, sl):
            for o, n in g_chunks:
                pltpu.async_copy(
                    table_hbm.at[ids_all.at[pl.ds(sl * L + o, n)]],
                    rows_v[b].at[pl.ds(o, n)],
                    sg[b],
                )

        def gather_wait(b):
            for o, n in g_chunks:
                pltpu.make_async_copy(
                    table_hbm.at[ids_all.at[pl.ds(o, n)]],
                    rows_v[b].at[pl.ds(o, n)],
                    sg[b],
                ).wait()

        def out_fire(b, sl):
            q = wid * steps + sl
            rbase = (q >> 5) * (16 * L) + (q & 15) * L
            hsel = (q >> 4) & 1
            pltpu.async_copy(
                out_v[b],
                out_hbm.at[pl.ds(rbase, L), pl.ds(hsel * E, E)],
                so[b],
            )

        def out_wait(b):
            pltpu.make_async_copy(
                out_v[b],
                out_hbm.at[pl.ds(0, L), pl.ds(0, E)],
                so[b],
            ).wait()

        def compute(b, sl):
            def do_rows(b16, j_lo):
                idvec = ids_all[pl.ds(sl * L + b16, LANES)]
                mvec = jnp.where(idvec == 0, 0.0, 1.0)
                for j in range(j_lo, LANES):
                    m = mvec[j]
                    w = b16 + j
                    for c in range(E // LANES):
                        sl16 = pl.ds(c * LANES, LANES)
                        out_v[b][w, sl16] = (
                            rows_v[b][w, sl16] * scale + pos_v[w, sl16]
                        ) * m

            @pl.loop(0, L // LANES)
            def _(g):
                do_rows(g * LANES, 0)

            if L % LANES:
                do_rows(L - LANES, LANES - L % LANES)

        gather_fire(0, 0)

        @pl.loop(0, steps // 2)
        def _(ss):
            for b in range(2):
                sl = ss * 2 + b

                @pl.when(sl + 1 < steps)
                def _():
                    gather_fire(1 - b, sl + 1)

                gather_wait(b)

                @pl.when(sl >= 2)
                def _():
                    out_wait(b)

                compute(b, sl)
                out_fire(b, sl)

        out_wait(0)
        out_wait(1)

    return k(token_table, ids, position_table)


def _tc_unpack(x128, B, L, E):
    G = 32
    nblk = B // G
    half = G // 2

    def body(in_ref, out_ref):
        x = in_ref[...]
        out_ref[0:half] = x[:, :E].reshape(half, L, E)
        out_ref[half:G] = x[:, E:].reshape(half, L, E)

    return pl.pallas_call(
        body,
        grid=(nblk,),
        in_specs=[pl.BlockSpec((half * L, 2 * E), lambda i: (i, 0))],
        out_specs=pl.BlockSpec((G, L, E), lambda i: (i, 0, 0)),
        out_shape=jax.ShapeDtypeStruct((B, L, E), jnp.float32),
    )(x128)


def kernel(inputs, token_table, position_table):
    B, L = inputs.shape
    V, E = token_table.shape
    ids = inputs.reshape(-1).astype(jnp.int32)
    out2 = _sc_embed(ids, token_table, position_table, B, L, E)
    return _tc_unpack(out2, B, L, E)
